# baseline (speedup 1.0000x reference)
import jax
import jax.numpy as jnp
from jax.experimental import pallas as pl
from jax.experimental.pallas import tpu as pltpu

_BATCH = 1024
_C = 100000
_SCALE = 64.0
_BASE_MARGIN = 0.5
_LAMBDA = 0.001
_G = 512
_NG = _BATCH // _G
_CH = 800
_NCH = _C // _CH
_NPAIR = _NCH // 2


def _loss_kernel(xt_ref, lg_ref, lgn_ref, lcol_ref, lrow_ref, out_ref,
                 buf, stg0, stg1, ssq_cur, xl_cur, ssq_nxt, xl_nxt,
                 smem, sems):
    g = pl.program_id(0)
    stgs = (stg0, stg1)

    def start_chunk(gi, cc, par):
        @pl.when(gi < _NG)
        def _():
            src = xt_ref.at[pl.ds(cc * _CH, _CH), pl.ds(gi * _G, _G)]
            pltpu.make_async_copy(src, stgs[par], sems.at[par]).start()

    def start_next(base_g, c, par):
        nc = c + 2
        start_chunk(base_g + nc // _NCH, nc % _NCH, par)

    def wait_chunk(par):
        pltpu.make_async_copy(
            xt_ref.at[pl.ds(0, _CH), pl.ds(0, _G)], stgs[par],
            sems.at[par]).wait()

    def refill_chunk(par, lbl, cc):
        xs = stgs[par][...]
        sn = jnp.sum(xs * xs, axis=0, keepdims=True)
        ridx = cc * _CH + jax.lax.broadcasted_iota(jnp.int32, (_CH, _G), 0)
        xn = jnp.sum(jnp.where(ridx == lbl, xs, 0.0), axis=0, keepdims=True)
        buf[pl.ds(cc * _CH, _CH), :] = xs.astype(jnp.float8_e4m3fn)
        ssq_nxt[...] += sn
        xl_nxt[...] += xn

    @pl.when(g == 0)
    def _():
        eq_all = (lcol_ref[...] == lrow_ref[...]).astype(jnp.float32)
        smem[0] = jnp.max(jnp.sum(eq_all, axis=0, keepdims=True))
        smem[1] = 0.0
        smem[2] = 0.0
        start_chunk(0, 0, 0)
        start_chunk(0, 1, 1)

        lbl0 = lg_ref[0]
        ssq_nxt[...] = jnp.zeros((1, _G), jnp.float32)
        xl_nxt[...] = jnp.zeros((1, _G), jnp.float32)

        def fill_pair(i, _):
            for off in (0, 1):
                c = 2 * i + off
                wait_chunk(off)
                refill_chunk(off, lbl0, c)
                start_next(0, c, off)
            return 0

        jax.lax.fori_loop(0, _NPAIR, fill_pair, 0)
        wait_chunk(0)
        refill_chunk(0, lbl0, _NCH - 1)
        start_next(0, _NCH - 1, 0)

    lg = lg_ref[0]
    lgn = lgn_ref[0]
    counts_g = jnp.sum((lcol_ref[...] == lg).astype(jnp.float32),
                       axis=0, keepdims=True)
    m = _BASE_MARGIN * smem[0] / counts_g

    ssq_cur[...] = ssq_nxt[...]
    xl_cur[...] = xl_nxt[...]
    ssq_nxt[...] = jnp.zeros((1, _G), jnp.float32)
    xl_nxt[...] = jnp.zeros((1, _G), jnp.float32)
    ssq = ssq_cur[...]
    xl = xl_cur[...]
    inv = 1.0 / jnp.maximum(jnp.sqrt(ssq), 1e-12)
    a = _SCALE * inv
    zero = jnp.zeros((1, _G), jnp.float32)

    def exp_chunk(c, se):
        xb = buf[pl.ds(c * _CH, _CH), :].astype(jnp.float32)
        return se + jnp.sum(jnp.exp(a * xb), axis=0, keepdims=True)

    has_next = g < _NG - 1

    def body_pair(i, se):
        for off in (0, 1):
            c = 2 * i + off
            se = exp_chunk(c, se)

            @pl.when(has_next)
            def _(off=off, c=c):
                par = (off + 1) % 2
                wait_chunk(par)
                refill_chunk(par, lgn, c)
                start_next(g + 1, c, par)
        return se

    se = jax.lax.fori_loop(0, _NPAIR, body_pair, zero)
    se = exp_chunk(_NCH - 1, se)

    @pl.when(has_next)
    def _():
        par = _NCH % 2
        wait_chunk(par)
        refill_chunk(par, lgn, _NCH - 1)
        start_next(g + 1, _NCH - 1, par)

    s_l = _SCALE * xl * inv
    r = jnp.exp(s_l) / se
    adj = jnp.maximum(r * (jnp.exp(-_SCALE * m) - 1.0), -1.0 + 1e-7)
    lse = jnp.log(se) + jnp.log1p(adj)
    true_logit = s_l - _SCALE * m

    smem[1] += jnp.sum(lse - true_logit)
    smem[2] += jnp.sum(m)

    @pl.when(g == _NG - 1)
    def _():
        total = smem[1] / _BATCH + _LAMBDA * (smem[2] / _BATCH)
        out_ref[...] = jnp.broadcast_to(total, (1, 1))


def kernel(logits, labels):
    xt = logits.T
    labg = labels.reshape(_NG, 1, _G)
    lcol = labels.reshape(_BATCH, 1)
    lrow = labels.reshape(1, _BATCH)
    loss = pl.pallas_call(
        _loss_kernel,
        grid=(_NG,),
        in_specs=[
            pl.BlockSpec(memory_space=pl.ANY),
            pl.BlockSpec((1, 1, _G), lambda g: (g, 0, 0)),
            pl.BlockSpec((1, 1, _G),
                         lambda g: (jnp.minimum(g + 1, _NG - 1), 0, 0)),
            pl.BlockSpec((_BATCH, 1), lambda g: (0, 0)),
            pl.BlockSpec((1, _BATCH), lambda g: (0, 0)),
        ],
        out_specs=pl.BlockSpec((1, 1), lambda g: (0, 0)),
        out_shape=jax.ShapeDtypeStruct((1, 1), jnp.float32),
        scratch_shapes=[
            pltpu.VMEM((_C, _G), jnp.float8_e4m3fn),
            pltpu.VMEM((_CH, _G), jnp.float32),
            pltpu.VMEM((_CH, _G), jnp.float32),
            pltpu.VMEM((1, _G), jnp.float32),
            pltpu.VMEM((1, _G), jnp.float32),
            pltpu.VMEM((1, _G), jnp.float32),
            pltpu.VMEM((1, _G), jnp.float32),
            pltpu.SMEM((4,), jnp.float32),
            pltpu.SemaphoreType.DMA((2,)),
        ],
        compiler_params=pltpu.CompilerParams(vmem_limit_bytes=64 * 1024 * 1024),
    )(xt, labg, labg, lcol, lrow)
    return loss[0, 0]


# speedup vs baseline: 1.1576x; 1.1576x over previous
import jax
import jax.numpy as jnp
from jax.experimental import pallas as pl
from jax.experimental.pallas import tpu as pltpu

_BATCH = 1024
_C = 100000
_SCALE = 64.0
_BASE_MARGIN = 0.5
_LAMBDA = 0.001
_G = 256
_NG = _BATCH // _G
_CH = 2000
_NCH = _C // _CH
_NPAIR = _NCH // 2
_SC = 400
_NSC = _CH // _SC


def _loss_kernel(xt_ref, lg_ref, lgn_ref, lcol_ref, lrow_ref, out_ref,
                 buf, stg0, stg1, ssq_cur, xl_cur, ssq_nxt, xl_nxt,
                 smem, sems):
    g = pl.program_id(0)
    stgs = (stg0, stg1)

    def start_chunk(gi, cc, par):
        @pl.when(gi < _NG)
        def _():
            src = xt_ref.at[pl.ds(cc * _CH, _CH), pl.ds(gi * _G, _G)]
            pltpu.make_async_copy(src, stgs[par], sems.at[par]).start()

    def start_next(base_g, c, par):
        nc = c + 2
        start_chunk(base_g + nc // _NCH, nc % _NCH, par)

    def wait_chunk(par):
        pltpu.make_async_copy(
            xt_ref.at[pl.ds(0, _CH), pl.ds(0, _G)], stgs[par],
            sems.at[par]).wait()

    def refill_chunk(par, lbl, cc):
        sn = jnp.zeros((1, _G), jnp.float32)
        xn = jnp.zeros((1, _G), jnp.float32)
        for s in range(_NSC):
            xs = stgs[par][pl.ds(s * _SC, _SC), :]
            sn = sn + jnp.sum(xs * xs, axis=0, keepdims=True)
            ridx = cc * _CH + s * _SC + jax.lax.broadcasted_iota(
                jnp.int32, (_SC, _G), 0)
            xn = xn + jnp.sum(jnp.where(ridx == lbl, xs, 0.0),
                              axis=0, keepdims=True)
            buf[pl.ds(cc * _CH + s * _SC, _SC), :] = xs.astype(jnp.bfloat16)
        ssq_nxt[...] += sn
        xl_nxt[...] += xn

    @pl.when(g == 0)
    def _():
        eq_all = (lcol_ref[...] == lrow_ref[...]).astype(jnp.float32)
        smem[0] = jnp.max(jnp.sum(eq_all, axis=0, keepdims=True))
        smem[1] = 0.0
        smem[2] = 0.0
        start_chunk(0, 0, 0)
        start_chunk(0, 1, 1)

        lbl0 = lg_ref[0]
        ssq_nxt[...] = jnp.zeros((1, _G), jnp.float32)
        xl_nxt[...] = jnp.zeros((1, _G), jnp.float32)

        def fill_pair(i, _):
            for off in (0, 1):
                c = 2 * i + off
                wait_chunk(off)
                refill_chunk(off, lbl0, c)
                start_next(0, c, off)
            return 0

        jax.lax.fori_loop(0, _NPAIR, fill_pair, 0)

    lg = lg_ref[0]
    lgn = lgn_ref[0]
    counts_g = jnp.sum((lcol_ref[...] == lg).astype(jnp.float32),
                       axis=0, keepdims=True)
    m = _BASE_MARGIN * smem[0] / counts_g

    ssq_cur[...] = ssq_nxt[...]
    xl_cur[...] = xl_nxt[...]
    ssq_nxt[...] = jnp.zeros((1, _G), jnp.float32)
    xl_nxt[...] = jnp.zeros((1, _G), jnp.float32)
    ssq = ssq_cur[...]
    xl = xl_cur[...]
    inv = 1.0 / jnp.maximum(jnp.sqrt(ssq), 1e-12)
    a = _SCALE * inv
    zero = jnp.zeros((1, _G), jnp.float32)

    def exp_chunk(c, se):
        for s in range(_NSC):
            xb = buf[pl.ds(c * _CH + s * _SC, _SC), :].astype(jnp.float32)
            se = se + jnp.sum(jnp.exp(a * xb), axis=0, keepdims=True)
        return se

    has_next = g < _NG - 1

    def body_pair(i, se):
        for off in (0, 1):
            c = 2 * i + off
            se = exp_chunk(c, se)

            @pl.when(has_next)
            def _(off=off, c=c):
                wait_chunk(off)
                refill_chunk(off, lgn, c)
                start_next(g + 1, c, off)
        return se

    se = jax.lax.fori_loop(0, _NPAIR, body_pair, zero)

    s_l = _SCALE * xl * inv
    r = jnp.exp(s_l) / se
    adj = jnp.maximum(r * (jnp.exp(-_SCALE * m) - 1.0), -1.0 + 1e-7)
    lse = jnp.log(se) + jnp.log1p(adj)
    true_logit = s_l - _SCALE * m

    smem[1] += jnp.sum(lse - true_logit)
    smem[2] += jnp.sum(m)

    @pl.when(g == _NG - 1)
    def _():
        total = smem[1] / _BATCH + _LAMBDA * (smem[2] / _BATCH)
        out_ref[...] = jnp.broadcast_to(total, (1, 1))


def kernel(logits, labels):
    xt = logits.T
    labg = labels.reshape(_NG, 1, _G)
    lcol = labels.reshape(_BATCH, 1)
    lrow = labels.reshape(1, _BATCH)
    loss = pl.pallas_call(
        _loss_kernel,
        grid=(_NG,),
        in_specs=[
            pl.BlockSpec(memory_space=pl.ANY),
            pl.BlockSpec((1, 1, _G), lambda g: (g, 0, 0)),
            pl.BlockSpec((1, 1, _G),
                         lambda g: (jnp.minimum(g + 1, _NG - 1), 0, 0)),
            pl.BlockSpec((_BATCH, 1), lambda g: (0, 0)),
            pl.BlockSpec((1, _BATCH), lambda g: (0, 0)),
        ],
        out_specs=pl.BlockSpec((1, 1), lambda g: (0, 0)),
        out_shape=jax.ShapeDtypeStruct((1, 1), jnp.float32),
        scratch_shapes=[
            pltpu.VMEM((_C, _G), jnp.bfloat16),
            pltpu.VMEM((_CH, _G), jnp.float32),
            pltpu.VMEM((_CH, _G), jnp.float32),
            pltpu.VMEM((1, _G), jnp.float32),
            pltpu.VMEM((1, _G), jnp.float32),
            pltpu.VMEM((1, _G), jnp.float32),
            pltpu.VMEM((1, _G), jnp.float32),
            pltpu.SMEM((4,), jnp.float32),
            pltpu.SemaphoreType.DMA((2,)),
        ],
        compiler_params=pltpu.CompilerParams(vmem_limit_bytes=64 * 1024 * 1024),
    )(xt, labg, labg, lcol, lrow)
    return loss[0, 0]
